# Initial kernel scaffold; baseline (speedup 1.0000x reference)
import jax
import jax.numpy as jnp
from jax.experimental import pallas as pl


def kernel(x, W_conv, W_fc, b_fc):
    raise NotImplementedError("write your pallas kernel here")


# speedup vs baseline: 28.0001x; 28.0001x over previous
import jax
import jax.numpy as jnp
from jax.experimental import pallas as pl
from jax.experimental.pallas import tpu as pltpu

B = 64
H = 512
W = 512
SIZE = 511
OUT = 128
T = 32
G = H // T


def _fused_kernel(wc_ref, x_ref, wfc_ref, b_ref, out_ref, carry_ref, acc_ref):
    i = pl.program_id(0)

    @pl.when(i == 0)
    def _init():
        carry_ref[...] = jnp.zeros_like(carry_ref)
        acc_ref[...] = jnp.zeros_like(acc_ref)

    wcv = wc_ref[...]
    w00 = wcv[0, 0]
    w01 = wcv[0, 1]
    w10 = wcv[0, 2]
    w11 = wcv[0, 3]

    carry_val = carry_ref[...]

    def row_contrib(r, bot):
        top = x_ref[:, r, :]
        y = jnp.tanh(w00 * top[:, :SIZE] + w01 * top[:, 1:]
                     + w10 * bot[:, :SIZE] + w11 * bot[:, 1:])
        w_r = wfc_ref[:, r, :]
        return jax.lax.dot_general(y, w_r, (((1,), (1,)), ((), ())),
                                   preferred_element_type=jnp.float32)

    acc = acc_ref[...]
    for r in range(T - 1):
        acc = acc + row_contrib(r, x_ref[:, r + 1, :])
    acc_ref[...] = acc

    @pl.when(i != 0)
    def _last_row():
        acc_ref[...] += row_contrib(T - 1, carry_val)

    carry_ref[...] = x_ref[:, 0, :]

    @pl.when(i == G - 1)
    def _finalize():
        out_ref[...] = jnp.tanh(acc_ref[...] + b_ref[...])


def kernel(x, W_conv, W_fc, b_fc):
    wfc3 = W_fc.reshape(OUT, SIZE, SIZE)
    wc = W_conv.reshape(1, 4)
    b2 = b_fc.reshape(1, OUT)
    return pl.pallas_call(
        _fused_kernel,
        grid=(G,),
        in_specs=[
            pl.BlockSpec((1, 4), lambda i: (0, 0)),
            pl.BlockSpec((B, T, W), lambda i: (0, G - 1 - i, 0)),
            pl.BlockSpec((OUT, T, SIZE), lambda i: (0, G - 1 - i, 0)),
            pl.BlockSpec((1, OUT), lambda i: (0, 0)),
        ],
        out_specs=pl.BlockSpec((B, OUT), lambda i: (0, 0)),
        out_shape=jax.ShapeDtypeStruct((B, OUT), jnp.float32),
        scratch_shapes=[
            pltpu.VMEM((B, W), jnp.float32),
            pltpu.VMEM((B, OUT), jnp.float32),
        ],
    )(wc, x, wfc3, b2)
